# baseline (speedup 1.0000x reference)
import jax
import jax.numpy as jnp
from jax import lax
from jax.experimental import pallas as pl

TOP_K = 64
NHIDDEN = 16
D = 512
N = 1194
GROUPS = 6
GLEN = 199


def _body(d1_ref, d2_ref, W_ref, W1_ref, b1_ref, W2_ref, b2_ref, out_ref):
    diff = d1_ref[...] - d2_ref[...]
    Et = lax.dot_general(
        W_ref[...], diff,
        dimension_numbers=(((0,), (1,)), ((), ())),
        preferred_element_type=jnp.float32,
    )
    Ee = Et + 1e-6
    s2 = jnp.sum(Ee * Ee, axis=0, keepdims=True)

    v = jnp.concatenate(
        [s2[:, g * GLEN:(g + 1) * GLEN] for g in range(GROUPS)], axis=0
    )

    col = lax.broadcasted_iota(jnp.int32, (GROUPS, GLEN), 1)
    kio = lax.broadcasted_iota(jnp.int32, (GROUPS, TOP_K), 1)

    def step(k, carry):
        v, xs = carry
        m = jnp.max(v, axis=1, keepdims=True)
        idx = jnp.min(jnp.where(v == m, col, GLEN), axis=1, keepdims=True)
        v = jnp.where(col == idx, -1.0, v)
        xs = xs + jnp.where(kio == k, m, 0.0)
        return v, xs

    xs0 = jnp.zeros((GROUPS, TOP_K), dtype=jnp.float32)
    _, xs = lax.fori_loop(0, TOP_K, step, (v, xs0))

    x = jnp.sqrt(xs)
    h = jnp.maximum(
        jnp.dot(x, W1_ref[...], preferred_element_type=jnp.float32)
        + b1_ref[...], 0.0)
    out_ref[...] = (
        jnp.dot(h, W2_ref[...], preferred_element_type=jnp.float32)
        + b2_ref[...])


def kernel(data1, data2, W_emb, b_emb, W1, b1, W2, b2):
    del b_emb
    out = pl.pallas_call(
        _body,
        out_shape=jax.ShapeDtypeStruct((GROUPS, 1), jnp.float32),
    )(data1, data2, W_emb, W1, b1.reshape(1, NHIDDEN), W2, b2.reshape(1, 1))
    return out


# speedup vs baseline: 2.1601x; 2.1601x over previous
import jax
import jax.numpy as jnp
from jax import lax
from jax.experimental import pallas as pl

TOP_K = 64
NHIDDEN = 16
D = 512
N = 1194
GROUPS = 6
GLEN = 199


def _body(d1_ref, d2_ref, W_ref, W1_ref, b1_ref, W2_ref, b2_ref, out_ref):
    f32 = jnp.float32
    diff = d1_ref[...] - d2_ref[...]
    Et = lax.dot_general(
        W_ref[...], diff,
        dimension_numbers=(((0,), (1,)), ((), ())),
        preferred_element_type=f32,
    )
    Ee = Et + 1e-6
    s2row = jnp.sum(Ee * Ee, axis=0, keepdims=True)

    eye = (lax.broadcasted_iota(jnp.int32, (GLEN, GLEN), 0)
           == lax.broadcasted_iota(jnp.int32, (GLEN, GLEN), 1)).astype(f32)
    subio = lax.broadcasted_iota(jnp.int32, (GLEN, GLEN), 0)
    lanio = lax.broadcasted_iota(jnp.int32, (GLEN, GLEN), 1)
    ones_col = jnp.ones((GLEN, 1), f32)
    kiof = lax.broadcasted_iota(jnp.int32, (GLEN, TOP_K), 1).astype(f32)

    xs_rows = []
    for g in range(GROUPS):
        rowg = s2row[:, g * GLEN:(g + 1) * GLEN]
        colg = lax.dot_general(
            eye, rowg, dimension_numbers=(((1,), (1,)), ((), ())),
            preferred_element_type=f32)
        gt = rowg > colg
        tie = (rowg == colg) & (lanio < subio)
        cnt = gt.astype(f32) + tie.astype(f32)
        rank = lax.dot_general(
            cnt, ones_col, dimension_numbers=(((1,), (0,)), ((), ())),
            preferred_element_type=f32)
        oh = (rank == kiof).astype(f32)
        xs_rows.append(lax.dot_general(
            rowg, oh, dimension_numbers=(((1,), (0,)), ((), ())),
            preferred_element_type=f32))

    xs = jnp.concatenate(xs_rows, axis=0)
    x = jnp.sqrt(xs)
    h = jnp.maximum(
        jnp.dot(x, W1_ref[...], preferred_element_type=f32)
        + b1_ref[...], 0.0)
    out_ref[...] = (
        jnp.dot(h, W2_ref[...], preferred_element_type=f32)
        + b2_ref[...])


def kernel(data1, data2, W_emb, b_emb, W1, b1, W2, b2):
    del b_emb
    out = pl.pallas_call(
        _body,
        out_shape=jax.ShapeDtypeStruct((GROUPS, 1), jnp.float32),
    )(data1, data2, W_emb, W1, b1.reshape(1, NHIDDEN), W2, b2.reshape(1, 1))
    return out
